# baseline (speedup 1.0000x reference)
import functools

import jax
import jax.numpy as jnp
from jax import lax
from jax.experimental import pallas as pl
from jax.experimental.pallas import tpu as pltpu, tpu_sc as plsc

RHO = 70.0
T, N = 1025, 20000
NPOS = 65536
SPAIRS = 100
TM1 = T - 1
TH = TM1 // 2
NW = 32
PP = NPOS // NW
RT = TH // NW
SS = RT * SPAIRS
L = 16


def _sigmoid(z):
    return 1.0 / (1.0 + jnp.exp(-z))


def _body(half, x_hbm, th_hbm, tp_hbm, up_hbm, vp_hbm, us_hbm, vs_hbm,
          kpos_hbm, kneg_hbm,
          th_v, tp_v, up_v, vp_v, iu_v, iv_v, gu_v, gv_v,
          su_v, sv_v, siu_v, siv_v, sgu_v, sgv_v, op_v, on_v, sem):
    wid = lax.axis_index("s") * 2 + lax.axis_index("c")

    pltpu.sync_copy(th_hbm, th_v)
    eps = _sigmoid(th_v[...]) * 0.5

    base = wid * PP
    pltpu.sync_copy(tp_hbm.at[pl.ds(base, PP)], tp_v)
    pltpu.sync_copy(up_hbm.at[pl.ds(base, PP)], up_v)
    pltpu.sync_copy(vp_hbm.at[pl.ds(base, PP)], vp_v)
    pltpu.sync_copy(us_hbm.at[wid], su_v)
    pltpu.sync_copy(vs_hbm.at[wid], sv_v)

    def pos_idx(k, c):
        s0 = pl.ds(2 * k * L, L)
        s1 = pl.ds((2 * k + 1) * L, L)
        lt0 = tp_v[s0] - half * TH
        lt1 = tp_v[s1] - half * TH
        ok0 = (lt0 >= 0) & (lt0 < TH)
        ok1 = (lt1 >= 0) & (lt1 < TH)
        r0 = lt0 * N
        r1 = lt1 * N
        iu_v[s0] = jnp.where(ok0, r0 + up_v[s0], 0)
        iv_v[s0] = jnp.where(ok0, r0 + vp_v[s0], 0)
        iu_v[s1] = jnp.where(ok1, r1 + up_v[s1], 0)
        iv_v[s1] = jnp.where(ok1, r1 + vp_v[s1], 0)
        return c
    lax.fori_loop(0, PP // L // 2, pos_idx, 0)

    iota = lax.iota(jnp.int32, L)
    t0 = (wid * RT + iota) * N

    def samp_idx(j2, c):
        b = j2 * 2 * L
        s0 = pl.ds(b, L)
        s1 = pl.ds(b + L, L)
        siu_v[s0] = su_v[s0] + t0
        siu_v[s1] = su_v[s1] + t0
        siv_v[s0] = sv_v[s0] + t0
        siv_v[s1] = sv_v[s1] + t0
        return c
    lax.fori_loop(0, SPAIRS // 2, samp_idx, 0)

    cu = pltpu.async_copy(x_hbm.at[iu_v], gu_v, sem)
    cv = pltpu.async_copy(x_hbm.at[iv_v], gv_v, sem)
    gsu = pltpu.async_copy(x_hbm.at[siu_v], sgu_v, sem)
    gsv = pltpu.async_copy(x_hbm.at[siv_v], sgv_v, sem)

    cu.wait()
    cv.wait()

    def pos_kap(k, c):
        s0 = pl.ds(2 * k * L, L)
        s1 = pl.ds((2 * k + 1) * L, L)
        d0 = gu_v[s0] - gv_v[s0]
        d1 = gu_v[s1] - gv_v[s1]
        op_v[s0] = _sigmoid(RHO * (eps - jnp.abs(d0)))
        op_v[s1] = _sigmoid(RHO * (eps - jnp.abs(d1)))
        return c
    lax.fori_loop(0, PP // L // 2, pos_kap, 0)
    pltpu.sync_copy(op_v, kpos_hbm.at[pl.ds(base, PP)])

    gsu.wait()
    gsv.wait()

    def samp_kap(j2, a0):
        b = j2 * 2 * L
        s0 = pl.ds(b, L)
        s1 = pl.ds(b + L, L)
        d0 = sgu_v[s0] - sgv_v[s0]
        d1 = sgu_v[s1] - sgv_v[s1]
        a0 = a0 + _sigmoid(RHO * (eps - jnp.abs(d0)))
        a0 = a0 + _sigmoid(RHO * (eps - jnp.abs(d1)))
        return a0
    zero = jnp.zeros((L,), jnp.float32)
    a0 = lax.fori_loop(0, SPAIRS // 2, samp_kap, zero)

    on_v[...] = 1.0 - a0 * (1.0 / SPAIRS)
    pltpu.sync_copy(on_v, kneg_hbm.at[pl.ds(wid * RT, RT)])


def _make_half(half):
    mesh = plsc.VectorSubcoreMesh(core_axis_name="c", subcore_axis_name="s")
    return pl.kernel(
        functools.partial(_body, half),
        out_type=(
            jax.ShapeDtypeStruct((NPOS,), jnp.float32),
            jax.ShapeDtypeStruct((TH,), jnp.float32),
        ),
        mesh=mesh,
        compiler_params=pltpu.CompilerParams(
            use_tc_tiling_on_sc=False, needs_layout_passes=False),
        scratch_types=[
            pltpu.VMEM((L,), jnp.float32),
            pltpu.VMEM((PP,), jnp.int32),
            pltpu.VMEM((PP,), jnp.int32),
            pltpu.VMEM((PP,), jnp.int32),
            pltpu.VMEM((PP,), jnp.int32),
            pltpu.VMEM((PP,), jnp.int32),
            pltpu.VMEM((PP,), jnp.float32),
            pltpu.VMEM((PP,), jnp.float32),
            pltpu.VMEM((SS,), jnp.int32),
            pltpu.VMEM((SS,), jnp.int32),
            pltpu.VMEM((SS,), jnp.int32),
            pltpu.VMEM((SS,), jnp.int32),
            pltpu.VMEM((SS,), jnp.float32),
            pltpu.VMEM((SS,), jnp.float32),
            pltpu.VMEM((PP,), jnp.float32),
            pltpu.VMEM((RT,), jnp.float32),
            pltpu.SemaphoreType.DMA,
        ],
    )


def _perm(s, half):
    blk = s[half * TH:(half + 1) * TH]
    return blk.reshape(NW, RT, SPAIRS).transpose(0, 2, 1).reshape(NW, SS)


def kernel(X, theta, u_pos, v_pos, t_pos, u_sample, v_sample):
    th16 = jnp.broadcast_to(theta.astype(jnp.float32), (L,))
    xa = X[:TH].reshape(-1)
    xb = X[TH:2 * TH].reshape(-1)

    run_a = _make_half(0)
    run_b = _make_half(1)
    kpos_a, kneg_a = run_a(xa, th16, t_pos, u_pos, v_pos,
                           _perm(u_sample, 0), _perm(v_sample, 0))
    kpos_b, kneg_b = run_b(xb, th16, t_pos, u_pos, v_pos,
                           _perm(u_sample, 1), _perm(v_sample, 1))

    kappa_pos = jnp.where(t_pos < TH, kpos_a, kpos_b)
    kappa_neg = jnp.concatenate([kneg_a, kneg_b])
    return kappa_pos, kappa_neg


# speedup vs baseline: 6.6948x; 6.6948x over previous
import jax
import jax.numpy as jnp
from jax import lax
from jax.experimental import pallas as pl
from jax.experimental.pallas import tpu as pltpu, tpu_sc as plsc

RHO = 70.0
T, N = 1025, 20000
NPOS = 65536
SPAIRS = 100
TM1 = T - 1
NW = 32
PP = NPOS // NW
RT = TM1 // NW
SS = RT * SPAIRS
L = 16


def _sigmoid(z):
    return 1.0 / (1.0 + jnp.exp(-z))


def _body(x_hbm, th_hbm, tp_hbm, up_hbm, vp_hbm, us_hbm, vs_hbm,
          kpos_hbm, kneg_hbm,
          th_v, tp_v, up_v, vp_v, iu_v, iv_v, gu_v, gv_v,
          su_v, sv_v, siu_v, siv_v, sgu_v, sgv_v, op_v, on_v, sem):
    wid = lax.axis_index("s") * 2 + lax.axis_index("c")

    pltpu.sync_copy(th_hbm, th_v)
    eps = _sigmoid(th_v[...]) * 0.5

    base = wid * PP
    pltpu.sync_copy(tp_hbm.at[pl.ds(base, PP)], tp_v)
    pltpu.sync_copy(up_hbm.at[pl.ds(base, PP)], up_v)
    pltpu.sync_copy(vp_hbm.at[pl.ds(base, PP)], vp_v)
    pltpu.sync_copy(us_hbm.at[wid], su_v)
    pltpu.sync_copy(vs_hbm.at[wid], sv_v)

    def pos_idx(k, c):
        s0 = pl.ds(2 * k * L, L)
        s1 = pl.ds((2 * k + 1) * L, L)
        r0 = tp_v[s0] * N
        r1 = tp_v[s1] * N
        iu_v[s0] = r0 + up_v[s0]
        iv_v[s0] = r0 + vp_v[s0]
        iu_v[s1] = r1 + up_v[s1]
        iv_v[s1] = r1 + vp_v[s1]
        return c
    lax.fori_loop(0, PP // L // 2, pos_idx, 0)

    iota = lax.iota(jnp.int32, L)
    t0 = (wid * RT + iota) * N
    t1 = (wid * RT + L + iota) * N

    def samp_idx(j2, c):
        b = j2 * 4 * L
        s0 = pl.ds(b, L)
        s1 = pl.ds(b + L, L)
        s2 = pl.ds(b + 2 * L, L)
        s3 = pl.ds(b + 3 * L, L)
        siu_v[s0] = su_v[s0] + t0
        siu_v[s1] = su_v[s1] + t1
        siv_v[s0] = sv_v[s0] + t0
        siv_v[s1] = sv_v[s1] + t1
        siu_v[s2] = su_v[s2] + t0
        siu_v[s3] = su_v[s3] + t1
        siv_v[s2] = sv_v[s2] + t0
        siv_v[s3] = sv_v[s3] + t1
        return c
    lax.fori_loop(0, SPAIRS // 2, samp_idx, 0)

    cu = pltpu.async_copy(x_hbm.at[iu_v], gu_v, sem)
    cv = pltpu.async_copy(x_hbm.at[iv_v], gv_v, sem)
    gsu = pltpu.async_copy(x_hbm.at[siu_v], sgu_v, sem)
    gsv = pltpu.async_copy(x_hbm.at[siv_v], sgv_v, sem)

    cu.wait()
    cv.wait()

    def pos_kap(k, c):
        s0 = pl.ds(2 * k * L, L)
        s1 = pl.ds((2 * k + 1) * L, L)
        d0 = gu_v[s0] - gv_v[s0]
        d1 = gu_v[s1] - gv_v[s1]
        op_v[s0] = _sigmoid(RHO * (eps - jnp.abs(d0)))
        op_v[s1] = _sigmoid(RHO * (eps - jnp.abs(d1)))
        return c
    lax.fori_loop(0, PP // L // 2, pos_kap, 0)
    pltpu.sync_copy(op_v, kpos_hbm.at[pl.ds(base, PP)])

    gsu.wait()
    gsv.wait()

    def samp_kap(j2, acc):
        a0, a1 = acc
        b = j2 * 4 * L
        s0 = pl.ds(b, L)
        s1 = pl.ds(b + L, L)
        s2 = pl.ds(b + 2 * L, L)
        s3 = pl.ds(b + 3 * L, L)
        d0 = sgu_v[s0] - sgv_v[s0]
        d1 = sgu_v[s1] - sgv_v[s1]
        d2 = sgu_v[s2] - sgv_v[s2]
        d3 = sgu_v[s3] - sgv_v[s3]
        a0 = a0 + _sigmoid(RHO * (eps - jnp.abs(d0)))
        a1 = a1 + _sigmoid(RHO * (eps - jnp.abs(d1)))
        a0 = a0 + _sigmoid(RHO * (eps - jnp.abs(d2)))
        a1 = a1 + _sigmoid(RHO * (eps - jnp.abs(d3)))
        return (a0, a1)
    zero = jnp.zeros((L,), jnp.float32)
    a0, a1 = lax.fori_loop(0, SPAIRS // 2, samp_kap, (zero, zero))

    on_v[pl.ds(0, L)] = 1.0 - a0 * (1.0 / SPAIRS)
    on_v[pl.ds(L, L)] = 1.0 - a1 * (1.0 / SPAIRS)
    pltpu.sync_copy(on_v, kneg_hbm.at[pl.ds(wid * RT, RT)])


def kernel(X, theta, u_pos, v_pos, t_pos, u_sample, v_sample):
    x_flat = X.reshape(-1)
    th16 = jnp.broadcast_to(theta.astype(jnp.float32), (L,))
    us_p = u_sample.reshape(NW, RT, SPAIRS).transpose(0, 2, 1).reshape(NW, SS)
    vs_p = v_sample.reshape(NW, RT, SPAIRS).transpose(0, 2, 1).reshape(NW, SS)

    mesh = plsc.VectorSubcoreMesh(core_axis_name="c", subcore_axis_name="s")
    run = pl.kernel(
        _body,
        out_type=(
            jax.ShapeDtypeStruct((NPOS,), jnp.float32),
            jax.ShapeDtypeStruct((TM1,), jnp.float32),
        ),
        mesh=mesh,
        compiler_params=pltpu.CompilerParams(
            use_tc_tiling_on_sc=False, needs_layout_passes=False),
        scratch_types=[
            pltpu.VMEM((L,), jnp.float32),
            pltpu.VMEM((PP,), jnp.int32),
            pltpu.VMEM((PP,), jnp.int32),
            pltpu.VMEM((PP,), jnp.int32),
            pltpu.VMEM((PP,), jnp.int32),
            pltpu.VMEM((PP,), jnp.int32),
            pltpu.VMEM((PP,), jnp.float32),
            pltpu.VMEM((PP,), jnp.float32),
            pltpu.VMEM((SS,), jnp.int32),
            pltpu.VMEM((SS,), jnp.int32),
            pltpu.VMEM((SS,), jnp.int32),
            pltpu.VMEM((SS,), jnp.int32),
            pltpu.VMEM((SS,), jnp.float32),
            pltpu.VMEM((SS,), jnp.float32),
            pltpu.VMEM((PP,), jnp.float32),
            pltpu.VMEM((RT,), jnp.float32),
            pltpu.SemaphoreType.DMA,
        ],
    )
    kappa_pos, kappa_neg = run(x_flat, th16, t_pos, u_pos, v_pos, us_p, vs_p)
    return kappa_pos, kappa_neg
